# baseline (speedup 1.0000x reference)
import functools

import jax
import jax.numpy as jnp
from jax import lax
from jax.experimental import pallas as pl
from jax.experimental.pallas import tpu as pltpu
from jax.experimental.pallas import tpu_sc as plsc

N = 65536
D = 256
NW = 32
CPW = D // NW
WIN = 8192
NVW = WIN // 16
NWINS = N // WIN
MININT = -(2 ** 31)


def _weights(n):
    grid = jnp.arange(1, n + 1, dtype=jnp.float32)
    pi = (grid - jnp.pi / 8.0) / (n + 0.25)
    m = jax.scipy.stats.norm.ppf(pi)
    return m / jnp.linalg.norm(m)


def _to_key(v):
    u = lax.bitcast_convert_type(v, jnp.int32)
    m = lax.shift_right_arithmetic(u, 31)
    return u ^ (m | MININT)


def _from_key(kk):
    top = lax.shift_right_logical(kk, 31)
    msk = MININT | (~(-top))
    return lax.bitcast_convert_type(kk ^ msk, jnp.float32)


@functools.partial(
    pl.kernel,
    mesh=plsc.VectorSubcoreMesh(core_axis_name="c", subcore_axis_name="s"),
    out_type=[
        jax.ShapeDtypeStruct((D, N), jnp.int32),
        jax.ShapeDtypeStruct((D, N), jnp.int32),
    ],
    scratch_types=[
        pltpu.VMEM((N,), jnp.int32),
        pltpu.VMEM((2048,), jnp.int32),
        pltpu.VMEM((2048,), jnp.int32),
        pltpu.VMEM((1024,), jnp.int32),
        pltpu.VMEM((WIN,), jnp.float32),
        pltpu.VMEM((WIN,), jnp.float32),
        pltpu.VMEM((WIN,), jnp.int32),
        pltpu.VMEM((WIN,), jnp.int32),
        pltpu.SemaphoreType.DMA,
        pltpu.SemaphoreType.DMA,
    ],
    compiler_params=pltpu.CompilerParams(needs_layout_passes=False),
)
def _sw_sc(xT, s1, s2,
           dest, histA, histB, histC,
           sf0, sf1, si0, si1, sem0, sem1):
    wid = lax.axis_index("c") * 16 + lax.axis_index("s")
    ones_i = jnp.ones((16,), jnp.int32)
    sems = (sem0, sem1)

    def streamed(src_slice, stages, body, carry_init):
        pltpu.async_copy(src_slice(0), stages[0], sems[0])

        def wb(w2, carry):
            for b in (0, 1):
                w = w2 * 2 + b
                nb = 1 - b

                @pl.when(w + 1 < NWINS)
                def _():
                    pltpu.async_copy(src_slice(w + 1), stages[nb], sems[nb])

                pltpu.make_async_copy(src_slice(w), stages[b], sems[b]).wait()
                carry = body(w, stages[b], carry)
            return carry
        return lax.fori_loop(0, NWINS // 2, wb, carry_init)

    def zero_hist(h, nbins):
        def zb(i, _):
            h[pl.ds(i * 16, 16)] = jnp.zeros((16,), jnp.int32)
            return 0
        lax.fori_loop(0, nbins // 16, zb, 0, unroll=4)

    def excl_prefix(h, nbins):
        def pb(i, carry):
            v = h[pl.ds(i * 16, 16)]
            inc = plsc.cumsum(v)
            h[pl.ds(i * 16, 16)] = inc - v + carry
            return carry + jnp.sum(v)
        lax.fori_loop(0, nbins // 16, pb, jnp.int32(0))

    def hist_all_body(w, stage, carry):
        @plsc.parallel_loop(0, NVW, step=1, unroll=4)
        def vb(j):
            kk = _to_key(stage[pl.ds(j * 16, 16)])
            plsc.addupdate_scatter(histA, [kk & jnp.int32(2047)], ones_i)
            plsc.addupdate_scatter(
                histB, [lax.shift_right_logical(kk, 11) & jnp.int32(2047)], ones_i)
            plsc.addupdate_scatter(
                histC, [lax.shift_right_logical(kk, 22) & jnp.int32(1023)], ones_i)
        return carry

    def make_permute_body(h, shift, bmask, from_f32):
        def body(w, stage, carry):
            def vb(j, _):
                if from_f32:
                    kk = _to_key(stage[pl.ds(j * 16, 16)])
                else:
                    kk = stage[pl.ds(j * 16, 16)]
                d = lax.shift_right_logical(kk, shift) & bmask
                cnt, last = plsc.scan_count(d)
                ofs = plsc.load_gather(h, [d])
                pos = ofs + cnt - 1
                plsc.store_scatter(dest, [pos], kk)
                plsc.addupdate_scatter(h, [d], cnt, mask=last)
                return 0
            lax.fori_loop(0, NVW, vb, 0, unroll=4)
            return carry
        return body

    def col_body(ci, _):
        c = wid * CPW + ci
        zero_hist(histA, 2048)
        zero_hist(histB, 2048)
        zero_hist(histC, 1024)
        xslice = lambda w: xT.at[c, pl.ds(w * WIN, WIN)]
        streamed(xslice, (sf0, sf1), hist_all_body, 0)
        excl_prefix(histA, 2048)
        excl_prefix(histB, 2048)
        excl_prefix(histC, 1024)
        streamed(xslice, (sf0, sf1),
                 make_permute_body(histA, 0, jnp.int32(2047), True), 0)
        pltpu.sync_copy(dest, s2.at[c])
        streamed(lambda w: s2.at[c, pl.ds(w * WIN, WIN)], (si0, si1),
                 make_permute_body(histB, 11, jnp.int32(2047), False), 0)
        pltpu.sync_copy(dest, s2.at[c])
        streamed(lambda w: s2.at[c, pl.ds(w * WIN, WIN)], (si0, si1),
                 make_permute_body(histC, 22, jnp.int32(1023), False), 0)
        pltpu.sync_copy(dest, s1.at[c])
        return 0

    lax.fori_loop(0, CPW, col_body, 0)


TCW = 4096


def _tc_dot(sorted_keys, k):
    g = N // TCW

    def body(s_ref, k_ref, num_ref, ss_ref):
        i = pl.program_id(0)

        @pl.when(i == 0)
        def _():
            num_ref[...] = jnp.zeros_like(num_ref)
            ss_ref[...] = jnp.zeros_like(ss_ref)

        v = _from_key(s_ref[...])
        kb = k_ref[...]
        num_ref[...] += jnp.sum(v * kb[None, :], axis=1)[None, :]
        ss_ref[...] += jnp.sum(v * v, axis=1)[None, :]

    return pl.pallas_call(
        body,
        grid=(g,),
        in_specs=[
            pl.BlockSpec((D, TCW), lambda i: (0, i)),
            pl.BlockSpec((TCW,), lambda i: (i,)),
        ],
        out_specs=[
            pl.BlockSpec((1, D), lambda i: (0, 0)),
            pl.BlockSpec((1, D), lambda i: (0, 0)),
        ],
        out_shape=[
            jax.ShapeDtypeStruct((1, D), jnp.float32),
            jax.ShapeDtypeStruct((1, D), jnp.float32),
        ],
    )(sorted_keys, k)


def kernel(x):
    eps = 1e-05
    n, d = x.shape
    k = lax.stop_gradient(_weights(n).astype(x.dtype))
    k_norm = jnp.linalg.norm(k)
    xT = x.T
    s_sorted, _ = _sw_sc(xT)
    num, ss = _tc_dot(s_sorted, k)
    s_norm = jnp.sqrt(ss[0])
    cos = num[0] / jnp.maximum(k_norm * s_norm, eps)
    return 1.0 - jnp.abs(cos)


# speedup vs baseline: 1.0054x; 1.0054x over previous
import functools

import jax
import jax.numpy as jnp
from jax import lax
from jax.experimental import pallas as pl
from jax.experimental.pallas import tpu as pltpu
from jax.experimental.pallas import tpu_sc as plsc

N = 65536
D = 256
NW = 32
CPW = D // NW
WIN = 4096
NVW = WIN // 16
NWINS = N // WIN
MININT = -(2 ** 31)


def _weights(n):
    grid = jnp.arange(1, n + 1, dtype=jnp.float32)
    pi = (grid - jnp.pi / 8.0) / (n + 0.25)
    m = jax.scipy.stats.norm.ppf(pi)
    return m / jnp.linalg.norm(m)


def _to_key(v):
    u = lax.bitcast_convert_type(v, jnp.int32)
    m = lax.shift_right_arithmetic(u, 31)
    return u ^ (m | MININT)


def _from_key(kk):
    top = lax.shift_right_logical(kk, 31)
    msk = MININT | (~(-top))
    return lax.bitcast_convert_type(kk ^ msk, jnp.float32)


@functools.partial(
    pl.kernel,
    mesh=plsc.VectorSubcoreMesh(core_axis_name="c", subcore_axis_name="s"),
    out_type=[
        jax.ShapeDtypeStruct((D, N), jnp.int32),
        jax.ShapeDtypeStruct((D, N), jnp.int32),
    ],
    scratch_types=[
        pltpu.VMEM((N,), jnp.int32),
        pltpu.VMEM((2048,), jnp.int32),
        pltpu.VMEM((2048,), jnp.int32),
        pltpu.VMEM((1024,), jnp.int32),
        pltpu.VMEM((WIN,), jnp.float32),
        pltpu.VMEM((WIN,), jnp.float32),
        pltpu.VMEM((WIN,), jnp.int32),
        pltpu.VMEM((WIN,), jnp.int32),
        pltpu.SemaphoreType.DMA,
        pltpu.SemaphoreType.DMA,
    ],
    compiler_params=pltpu.CompilerParams(needs_layout_passes=False),
)
def _sw_sc(xT, s1, s2,
           dest, histA, histB, histC,
           sf0, sf1, si0, si1, sem0, sem1):
    wid = lax.axis_index("c") * 16 + lax.axis_index("s")
    ones_i = jnp.ones((16,), jnp.int32)
    sems = (sem0, sem1)

    def streamed(src_slice, stages, body, carry_init):
        pltpu.async_copy(src_slice(0), stages[0], sems[0])

        def wb(w2, carry):
            for b in (0, 1):
                w = w2 * 2 + b
                nb = 1 - b

                @pl.when(w + 1 < NWINS)
                def _():
                    pltpu.async_copy(src_slice(w + 1), stages[nb], sems[nb])

                pltpu.make_async_copy(src_slice(w), stages[b], sems[b]).wait()
                carry = body(w, stages[b], carry)
            return carry
        return lax.fori_loop(0, NWINS // 2, wb, carry_init)

    def zero_hist(h, nbins):
        def zb(i, _):
            h[pl.ds(i * 16, 16)] = jnp.zeros((16,), jnp.int32)
            return 0
        lax.fori_loop(0, nbins // 16, zb, 0, unroll=4)

    def excl_prefix(h, nbins):
        def pb(i, carry):
            v = h[pl.ds(i * 16, 16)]
            inc = plsc.cumsum(v)
            h[pl.ds(i * 16, 16)] = inc - v + carry
            return carry + jnp.sum(v)
        lax.fori_loop(0, nbins // 16, pb, jnp.int32(0))

    def hist_all_body(w, stage, carry):
        @plsc.parallel_loop(0, NVW, step=1, unroll=4)
        def vb(j):
            kk = _to_key(stage[pl.ds(j * 16, 16)])
            plsc.addupdate_scatter(histA, [kk & jnp.int32(2047)], ones_i)
            plsc.addupdate_scatter(
                histB, [lax.shift_right_logical(kk, 11) & jnp.int32(2047)], ones_i)
            plsc.addupdate_scatter(
                histC, [lax.shift_right_logical(kk, 22) & jnp.int32(1023)], ones_i)
        return carry

    def make_permute_body(h, shift, bmask, from_f32):
        def body(w, stage, carry):
            def vb(j, _):
                if from_f32:
                    kk = _to_key(stage[pl.ds(j * 16, 16)])
                else:
                    kk = stage[pl.ds(j * 16, 16)]
                d = lax.shift_right_logical(kk, shift) & bmask
                cnt, last = plsc.scan_count(d)
                ofs = plsc.load_gather(h, [d])
                pos = ofs + cnt - 1
                plsc.store_scatter(dest, [pos], kk)
                plsc.addupdate_scatter(h, [d], cnt, mask=last)
                return 0
            lax.fori_loop(0, NVW, vb, 0, unroll=8)
            return carry
        return body

    def col_body(ci, _):
        c = wid * CPW + ci
        zero_hist(histA, 2048)
        zero_hist(histB, 2048)
        zero_hist(histC, 1024)
        xslice = lambda w: xT.at[c, pl.ds(w * WIN, WIN)]
        streamed(xslice, (sf0, sf1), hist_all_body, 0)
        excl_prefix(histA, 2048)
        excl_prefix(histB, 2048)
        excl_prefix(histC, 1024)
        streamed(xslice, (sf0, sf1),
                 make_permute_body(histA, 0, jnp.int32(2047), True), 0)
        pltpu.sync_copy(dest, s2.at[c])
        streamed(lambda w: s2.at[c, pl.ds(w * WIN, WIN)], (si0, si1),
                 make_permute_body(histB, 11, jnp.int32(2047), False), 0)
        pltpu.sync_copy(dest, s2.at[c])
        streamed(lambda w: s2.at[c, pl.ds(w * WIN, WIN)], (si0, si1),
                 make_permute_body(histC, 22, jnp.int32(1023), False), 0)
        pltpu.sync_copy(dest, s1.at[c])
        return 0

    lax.fori_loop(0, CPW, col_body, 0)


TCW = 4096


def _tc_dot(sorted_keys, k):
    g = N // TCW

    def body(s_ref, k_ref, num_ref, ss_ref):
        i = pl.program_id(0)

        @pl.when(i == 0)
        def _():
            num_ref[...] = jnp.zeros_like(num_ref)
            ss_ref[...] = jnp.zeros_like(ss_ref)

        v = _from_key(s_ref[...])
        kb = k_ref[...]
        num_ref[...] += jnp.sum(v * kb[None, :], axis=1)[None, :]
        ss_ref[...] += jnp.sum(v * v, axis=1)[None, :]

    return pl.pallas_call(
        body,
        grid=(g,),
        in_specs=[
            pl.BlockSpec((D, TCW), lambda i: (0, i)),
            pl.BlockSpec((TCW,), lambda i: (i,)),
        ],
        out_specs=[
            pl.BlockSpec((1, D), lambda i: (0, 0)),
            pl.BlockSpec((1, D), lambda i: (0, 0)),
        ],
        out_shape=[
            jax.ShapeDtypeStruct((1, D), jnp.float32),
            jax.ShapeDtypeStruct((1, D), jnp.float32),
        ],
    )(sorted_keys, k)


def kernel(x):
    eps = 1e-05
    n, d = x.shape
    k = lax.stop_gradient(_weights(n).astype(x.dtype))
    k_norm = jnp.linalg.norm(k)
    xT = x.T
    s_sorted, _ = _sw_sc(xT)
    num, ss = _tc_dot(s_sorted, k)
    s_norm = jnp.sqrt(ss[0])
    cos = num[0] / jnp.maximum(k_norm * s_norm, eps)
    return 1.0 - jnp.abs(cos)
